# baseline (speedup 1.0000x reference)
import functools
import jax
import jax.numpy as jnp
from jax import lax
from jax.experimental import pallas as pl
from jax.experimental.pallas import tpu as pltpu, tpu_sc as plsc

_REP = 2
_SH_ROWS = 48
_STREAM_ROWS = 80
_DMA_ROWS = 48


def kernel(x, table):
    batch, seq_len = x.shape
    _, dim = table.shape
    width = seq_len * dim
    flat = table[:seq_len].reshape(width)

    info = plsc.get_sparse_core_info()
    nc, ns = info.num_cores, info.num_subcores
    rows_per_sc = batch // nc
    assert _STREAM_ROWS + _DMA_ROWS == rows_per_sc // ns
    fill = _SH_ROWS // ns

    mesh = plsc.VectorSubcoreMesh(core_axis_name="c", subcore_axis_name="s")

    @functools.partial(
        pl.kernel,
        mesh=mesh,
        out_type=jax.ShapeDtypeStruct((batch, width), jnp.float32),
        scratch_types=[
            pltpu.VMEM((_REP, width), jnp.float32),
            pltpu.VMEM_SHARED((_SH_ROWS, width), jnp.float32),
            pltpu.SemaphoreType.DMA,
            pltpu.SemaphoreType.DMA,
        ],
    )
    def bcast(table_hbm, out_hbm, buf, shared, sem_st, sem_dm):
        c = lax.axis_index("c")
        s = lax.axis_index("s")
        for r in range(_REP):
            pltpu.sync_copy(table_hbm, buf.at[r])
        for r in range(fill):
            pltpu.sync_copy(table_hbm, shared.at[s * fill + r])
        plsc.subcore_barrier()
        st_base = c * rows_per_sc + s * _STREAM_ROWS
        dm_base = c * rows_per_sc + ns * _STREAM_ROWS + s * _DMA_ROWS
        copies = []
        for k in range(_DMA_ROWS // _SH_ROWS):
            copies.append(pltpu.async_copy(
                shared, out_hbm.at[pl.ds(dm_base + k * _SH_ROWS, _SH_ROWS)], sem_dm))
        for k in range(_STREAM_ROWS // _REP):
            copies.append(pltpu.async_copy(
                buf, out_hbm.at[pl.ds(st_base + k * _REP, _REP)], sem_st))
        for cp in copies:
            cp.wait()

    out = bcast(flat)
    return out.reshape(batch, seq_len, dim)


# speedup vs baseline: 1.0196x; 1.0196x over previous
import functools
import jax
import jax.numpy as jnp
from jax import lax
from jax.experimental import pallas as pl
from jax.experimental.pallas import tpu as pltpu, tpu_sc as plsc

_REP = 2
_SH_ROWS = 16
_STREAM_ROWS = 80
_DMA_ROWS = 48


def kernel(x, table):
    batch, seq_len = x.shape
    _, dim = table.shape
    width = seq_len * dim
    flat = table[:seq_len].reshape(width)

    info = plsc.get_sparse_core_info()
    nc, ns = info.num_cores, info.num_subcores
    rows_per_sc = batch // nc
    assert _STREAM_ROWS + _DMA_ROWS == rows_per_sc // ns
    fill = _SH_ROWS // ns

    mesh = plsc.VectorSubcoreMesh(core_axis_name="c", subcore_axis_name="s")

    @functools.partial(
        pl.kernel,
        mesh=mesh,
        out_type=jax.ShapeDtypeStruct((batch, width), jnp.float32),
        scratch_types=[
            pltpu.VMEM((_REP, width), jnp.float32),
            pltpu.VMEM_SHARED((_SH_ROWS, width), jnp.float32),
            pltpu.SemaphoreType.DMA,
            pltpu.SemaphoreType.DMA,
        ],
    )
    def bcast(table_hbm, out_hbm, buf, shared, sem_st, sem_dm):
        c = lax.axis_index("c")
        s = lax.axis_index("s")
        for r in range(_REP):
            pltpu.sync_copy(table_hbm, buf.at[r])
        for r in range(fill):
            pltpu.sync_copy(table_hbm, shared.at[s * fill + r])
        plsc.subcore_barrier()
        st_base = c * rows_per_sc + s * _STREAM_ROWS
        dm_base = c * rows_per_sc + ns * _STREAM_ROWS + s * _DMA_ROWS
        copies = []
        for k in range(_DMA_ROWS // _SH_ROWS):
            copies.append(pltpu.async_copy(
                shared, out_hbm.at[pl.ds(dm_base + k * _SH_ROWS, _SH_ROWS)], sem_dm))
        for k in range(_STREAM_ROWS // _REP):
            copies.append(pltpu.async_copy(
                buf, out_hbm.at[pl.ds(st_base + k * _REP, _REP)], sem_st))
        for cp in copies:
            cp.wait()

    out = bcast(flat)
    return out.reshape(batch, seq_len, dim)
